# Initial kernel scaffold; baseline (speedup 1.0000x reference)
import jax
import jax.numpy as jnp
from jax.experimental import pallas as pl


def kernel(bev_feats, prior_feats, prior_voxels_coords, w1, b1, w2, b2, conv1_w, conv1_b, bn1_g, bn1_b, conv2_w, conv2_b, bn2_g, bn2_b, wq, bq, wk, bk, wv, bv, wo, bo, win_w, win_b, wout_w, wout_b):
    raise NotImplementedError("write your pallas kernel here")


# speedup vs baseline: 1.0322x; 1.0322x over previous
import jax
import jax.numpy as jnp
import numpy as np
from jax.experimental import pallas as pl
from jax.experimental.pallas import tpu as pltpu

BS = 1; X = 400; Y = 400; ZP = 8; CV = 64; H = 200; W = 200; BZ = 8; BC = 80
HID = 256; NW = 10; WS = 20; NH = 8; HD = 32; N = 100000; CIN = 68; EPS = 1e-5
NPIX = X * Y


def _mlp_body(p_ref, w1_ref, b1_ref, w2_ref, b2_ref, f_ref):
    h = jnp.dot(p_ref[...], w1_ref[...], preferred_element_type=jnp.float32)
    h = jnp.maximum(h + b1_ref[0], 0.0)
    o = jnp.dot(h, w2_ref[...], preferred_element_type=jnp.float32)
    f_ref[...] = jnp.maximum(o + b2_ref[0], 0.0)


def _mlp(p, w1, b1, w2, b2):
    BM = 2000
    g = N // BM
    return pl.pallas_call(
        _mlp_body,
        grid=(g,),
        in_specs=[
            pl.BlockSpec((BM, CIN), lambda i: (i, 0)),
            pl.BlockSpec((CIN, CV), lambda i: (0, 0)),
            pl.BlockSpec((1, CV), lambda i: (0, 0)),
            pl.BlockSpec((CV, CV), lambda i: (0, 0)),
            pl.BlockSpec((1, CV), lambda i: (0, 0)),
        ],
        out_specs=pl.BlockSpec((BM, CV), lambda i: (i, 0)),
        out_shape=jax.ShapeDtypeStruct((N, CV), jnp.float32),
        compiler_params=pltpu.CompilerParams(
            dimension_semantics=("parallel",)),
    )(p, w1, b1[None], w2, b2[None])


def _c1_body(g_ref, w_ref, b_ref, out_ref, st_ref):
    o = jnp.dot(g_ref[...], w_ref[...], preferred_element_type=jnp.float32)
    o = o + b_ref[0]
    out_ref[...] = o
    st_ref[0, 0, :] = jnp.sum(o, axis=0)
    st_ref[0, 1, :] = jnp.sum(o * o, axis=0)


def _conv1(g, w, b):
    BM = 2000
    gr = NPIX // BM
    return pl.pallas_call(
        _c1_body,
        grid=(gr,),
        in_specs=[
            pl.BlockSpec((BM, CV * ZP), lambda i: (i, 0)),
            pl.BlockSpec((CV * ZP, HID), lambda i: (0, 0)),
            pl.BlockSpec((1, HID), lambda i: (0, 0)),
        ],
        out_specs=[
            pl.BlockSpec((BM, HID), lambda i: (i, 0)),
            pl.BlockSpec((1, 2, HID), lambda i: (i, 0, 0)),
        ],
        out_shape=[
            jax.ShapeDtypeStruct((NPIX, HID), jnp.float32),
            jax.ShapeDtypeStruct((gr, 2, HID), jnp.float32),
        ],
        compiler_params=pltpu.CompilerParams(
            dimension_semantics=("parallel",)),
    )(g, w, b[None])


def _c2_body(prev_ref, cur_ref, next_ref, sc_ref, sh_ref, w_ref, b_ref,
             out_ref, st_ref):
    i = pl.program_id(0)
    nstrips = pl.num_programs(0)
    buf = jnp.concatenate(
        [prev_ref[7:8], cur_ref[...], next_ref[0:1]], axis=0)
    act = jnp.maximum(buf * sc_ref[0] + sh_ref[0], 0.0)
    y0 = i * 8 - 1
    rows = jax.lax.broadcasted_iota(jnp.int32, (10, 1, 1), 0) + y0
    act = jnp.where((rows >= 0) & (rows < Y), act, 0.0)
    padded = jnp.pad(act, ((0, 0), (1, 1), (0, 0)))
    acc = jnp.zeros((8 * X, HID), jnp.float32) + b_ref[0]
    for k in range(9):
        ky, kx = k // 3, k % 3
        patch = padded[ky:ky + 8, kx:kx + X, :].reshape(8 * X, HID)
        acc = acc + jnp.dot(patch, w_ref[k],
                            preferred_element_type=jnp.float32)
    out_ref[...] = acc.reshape(8, X, HID)
    st_ref[0, 0, :] = jnp.sum(acc, axis=0)
    st_ref[0, 1, :] = jnp.sum(acc * acc, axis=0)
    del nstrips


def _conv2(x, sc, sh, w, b):
    gr = Y // 8
    blk = pl.BlockSpec((8, X, HID), lambda i: (i, 0, 0))
    return pl.pallas_call(
        _c2_body,
        grid=(gr,),
        in_specs=[
            pl.BlockSpec((8, X, HID), lambda i: (jnp.maximum(i - 1, 0), 0, 0)),
            blk,
            pl.BlockSpec((8, X, HID),
                         lambda i: (jnp.minimum(i + 1, gr - 1), 0, 0)),
            pl.BlockSpec((1, HID), lambda i: (0, 0)),
            pl.BlockSpec((1, HID), lambda i: (0, 0)),
            pl.BlockSpec((9, HID, HID), lambda i: (0, 0, 0)),
            pl.BlockSpec((1, HID), lambda i: (0, 0)),
        ],
        out_specs=[
            pl.BlockSpec((8, X, HID), lambda i: (i, 0, 0)),
            pl.BlockSpec((1, 2, HID), lambda i: (i, 0, 0)),
        ],
        out_shape=[
            jax.ShapeDtypeStruct((Y, X, HID), jnp.float32),
            jax.ShapeDtypeStruct((gr, 2, HID), jnp.float32),
        ],
        compiler_params=pltpu.CompilerParams(
            dimension_semantics=("arbitrary",)),
    )(x, x, x, sc[None], sh[None], w, b[None])


def _attn_body(x2_ref, sc_ref, sh_ref, bev_ref, winw_ref, winb_ref,
               wq_ref, bq_ref, wk_ref, bk_ref, wv_ref, bv_ref,
               wo_ref, bo_ref, wow_ref, wob_ref, out_ref):
    t = jnp.maximum(x2_ref[...] * sc_ref[0] + sh_ref[0], 0.0)
    t = t.reshape(20, 2, 40, HID).max(axis=1)
    t = t.reshape(20, 20, 2, HID).max(axis=2)
    kv = t.reshape(WS * WS, HID)

    bev = bev_ref[0]
    qw = jnp.dot(bev, winw_ref[...],
                 preferred_element_type=jnp.float32) + winb_ref[0]
    qh = jnp.dot(qw, wq_ref[...], preferred_element_type=jnp.float32) + bq_ref[0]
    kh = jnp.dot(kv, wk_ref[...], preferred_element_type=jnp.float32) + bk_ref[0]
    vh = jnp.dot(kv, wv_ref[...], preferred_element_type=jnp.float32) + bv_ref[0]
    scale = jnp.float32(1.0 / np.sqrt(HD))
    outs = []
    for h in range(NH):
        s = slice(h * HD, (h + 1) * HD)
        a = jax.lax.dot_general(qh[:, s], kh[:, s],
                                (((1,), (1,)), ((), ())),
                                preferred_element_type=jnp.float32) * scale
        a = jax.nn.softmax(a, axis=-1)
        outs.append(jnp.dot(a, vh[:, s], preferred_element_type=jnp.float32))
    o = jnp.concatenate(outs, axis=-1)
    o = jnp.dot(o, wo_ref[...], preferred_element_type=jnp.float32) \
        + bo_ref[0] + qw
    res = jnp.dot(o, wow_ref[...],
                  preferred_element_type=jnp.float32) + wob_ref[0]
    out_ref[0] = res


def _attention(x2, sc, sh, bev_win, win_w, win_b, wq, bq, wk, bk, wv, bv,
               wo, bo, wout_w, wout_b):
    full2 = lambda a, b: pl.BlockSpec((a, b), lambda wy, wx: (0, 0))
    return pl.pallas_call(
        _attn_body,
        grid=(NW, NW),
        in_specs=[
            pl.BlockSpec((2 * WS, 2 * WS, HID), lambda wy, wx: (wy, wx, 0)),
            full2(1, HID), full2(1, HID),
            pl.BlockSpec((1, WS * WS, BZ * BC),
                         lambda wy, wx: (wy * NW + wx, 0, 0)),
            full2(BZ * BC, HID), full2(1, HID),
            full2(HID, HID), full2(1, HID),
            full2(HID, HID), full2(1, HID),
            full2(HID, HID), full2(1, HID),
            full2(HID, HID), full2(1, HID),
            full2(HID, BZ * BC), full2(1, BZ * BC),
        ],
        out_specs=pl.BlockSpec((1, WS * WS, BZ * BC),
                               lambda wy, wx: (wy * NW + wx, 0, 0)),
        out_shape=jax.ShapeDtypeStruct((NW * NW, WS * WS, BZ * BC),
                                       jnp.float32),
        compiler_params=pltpu.CompilerParams(
            dimension_semantics=("parallel", "arbitrary")),
    )(x2, sc[None], sh[None], bev_win, win_w, win_b[None], wq, bq[None],
      wk, bk[None], wv, bv[None], wo, bo[None], wout_w, wout_b[None])


def _finalize_bn(stats, g, b):
    s = jnp.sum(stats[:, 0, :], axis=0)
    sq = jnp.sum(stats[:, 1, :], axis=0)
    m = s / NPIX
    v = sq / NPIX - m * m
    sc = g * jax.lax.rsqrt(v + EPS)
    return sc, b - m * sc


def kernel(bev_feats, prior_feats, prior_voxels_coords, w1, b1, w2, b2,
           conv1_w, conv1_b, bn1_g, bn1_b, conv2_w, conv2_b, bn2_g, bn2_b,
           wq, bq, wk, bk, wv, bv, wo, bo, win_w, win_b, wout_w, wout_b):
    f = _mlp(prior_feats.reshape(N, CIN), w1, b1, w2, b2)

    co = prior_voxels_coords.reshape(N, 3)
    vox = jnp.zeros((Y, X, ZP, CV), jnp.float32)
    vox = vox.at[co[:, 1], co[:, 0], co[:, 2]].set(f)
    g = vox.reshape(NPIX, ZP * CV)

    w1x1 = conv1_w[:, :, 0, 0].reshape(HID, CV, ZP)
    w1x1 = w1x1.transpose(2, 1, 0).reshape(ZP * CV, HID)
    out1, st1 = _conv1(g, w1x1, conv1_b)
    sc1, sh1 = _finalize_bn(st1, bn1_g, bn1_b)

    w3 = conv2_w.transpose(2, 3, 1, 0).reshape(9, HID, HID)
    out2, st2 = _conv2(out1.reshape(Y, X, HID), sc1, sh1, w3, conv2_b)
    sc2, sh2 = _finalize_bn(st2, bn2_g, bn2_b)

    bev = bev_feats.reshape(BC, H, W, BZ).transpose(1, 2, 3, 0)
    bev = bev.reshape(NW, WS, NW, WS, BZ * BC).transpose(0, 2, 1, 3, 4)
    bev_win = bev.reshape(NW * NW, WS * WS, BZ * BC)

    outw = _attention(out2, sc2, sh2, bev_win, win_w, win_b,
                      wq, bq, wk, bk, wv, bv, wo, bo, wout_w, wout_b)

    out = outw.reshape(NW, NW, WS, WS, BZ * BC).transpose(0, 2, 1, 3, 4)
    out = out.reshape(H, W, BZ, BC).transpose(3, 0, 1, 2)
    return out[None]


# speedup vs baseline: 1.4704x; 1.4245x over previous
import jax
import jax.numpy as jnp
import numpy as np
from jax.experimental import pallas as pl
from jax.experimental.pallas import tpu as pltpu

BS = 1; X = 400; Y = 400; ZP = 8; CV = 64; H = 200; W = 200; BZ = 8; BC = 80
HID = 256; NW = 10; WS = 20; NH = 8; HD = 32; N = 100000; CIN = 68; EPS = 1e-5
NPIX = X * Y


def _mlp_body(p_ref, w1_ref, b1_ref, w2_ref, b2_ref, f_ref):
    h = jnp.dot(p_ref[...], w1_ref[...], preferred_element_type=jnp.float32)
    h = jnp.maximum(h + b1_ref[0], 0.0)
    o = jnp.dot(h, w2_ref[...], preferred_element_type=jnp.float32)
    f_ref[...] = jnp.maximum(o + b2_ref[0], 0.0)


def _mlp(p, w1, b1, w2, b2):
    BM = 2000
    g = N // BM
    return pl.pallas_call(
        _mlp_body,
        grid=(g,),
        in_specs=[
            pl.BlockSpec((BM, CIN), lambda i: (i, 0)),
            pl.BlockSpec((CIN, CV), lambda i: (0, 0)),
            pl.BlockSpec((1, CV), lambda i: (0, 0)),
            pl.BlockSpec((CV, CV), lambda i: (0, 0)),
            pl.BlockSpec((1, CV), lambda i: (0, 0)),
        ],
        out_specs=pl.BlockSpec((BM, CV), lambda i: (i, 0)),
        out_shape=jax.ShapeDtypeStruct((N, CV), jnp.float32),
        compiler_params=pltpu.CompilerParams(
            dimension_semantics=("parallel",)),
    )(p, w1, b1[None], w2, b2[None])


NSTRIP = 50
SVOX = (Y // NSTRIP) * X * ZP


def _sc1_body(starts_ref, packed_ref, f_ref, w_ref, b_ref,
              out_ref, st_ref, grid_ref):
    i = pl.program_id(0)
    lo = starts_ref[i]
    hi = starts_ref[i + 1]
    grid_ref[...] = jnp.zeros((SVOX, CV), jnp.float32)

    def body(j, carry):
        pk = packed_ref[j]
        vloc = jax.lax.shift_right_logical(pk, 17)
        pj = pk & 0x1FFFF
        row = f_ref[pl.ds(jax.lax.shift_right_logical(pj, 1), 1), :]
        chunk = jnp.where((pj & 1) == 0, row[:, 0:CV], row[:, CV:2 * CV])
        grid_ref[pl.ds(vloc, 1), :] = chunk
        return carry

    jax.lax.fori_loop(lo, hi, body, 0)

    g3 = grid_ref[...].reshape(SVOX // ZP, ZP, CV)
    acc = jnp.zeros((SVOX // ZP, HID), jnp.float32) + b_ref[0]
    for z in range(ZP):
        acc = acc + jnp.dot(g3[:, z, :], w_ref[z],
                            preferred_element_type=jnp.float32)
    out_ref[...] = acc
    st_ref[0, 0, :] = jnp.sum(acc, axis=0)
    st_ref[0, 1, :] = jnp.sum(acc * acc, axis=0)


def _scatter_conv1(packed, starts, f_pairs, w, b):
    return pl.pallas_call(
        _sc1_body,
        grid=(NSTRIP,),
        in_specs=[
            pl.BlockSpec(memory_space=pltpu.SMEM),
            pl.BlockSpec(memory_space=pltpu.SMEM),
            pl.BlockSpec((N // 2, 2 * CV), lambda i: (0, 0)),
            pl.BlockSpec((ZP, CV, HID), lambda i: (0, 0, 0)),
            pl.BlockSpec((1, HID), lambda i: (0, 0)),
        ],
        out_specs=[
            pl.BlockSpec((SVOX // ZP, HID), lambda i: (i, 0)),
            pl.BlockSpec((1, 2, HID), lambda i: (i, 0, 0)),
        ],
        out_shape=[
            jax.ShapeDtypeStruct((NPIX, HID), jnp.float32),
            jax.ShapeDtypeStruct((NSTRIP, 2, HID), jnp.float32),
        ],
        scratch_shapes=[pltpu.VMEM((SVOX, CV), jnp.float32)],
        compiler_params=pltpu.CompilerParams(
            dimension_semantics=("arbitrary",)),
    )(starts, packed, f_pairs, w, b[None])


def _c2_body(prev_ref, cur_ref, next_ref, sc_ref, sh_ref, w_ref, b_ref,
             out_ref, st_ref):
    i = pl.program_id(0)
    nstrips = pl.num_programs(0)
    buf = jnp.concatenate(
        [prev_ref[7:8], cur_ref[...], next_ref[0:1]], axis=0)
    act = jnp.maximum(buf * sc_ref[0] + sh_ref[0], 0.0)
    y0 = i * 8 - 1
    rows = jax.lax.broadcasted_iota(jnp.int32, (10, 1, 1), 0) + y0
    act = jnp.where((rows >= 0) & (rows < Y), act, 0.0)
    padded = jnp.pad(act, ((0, 0), (1, 1), (0, 0)))
    acc = jnp.zeros((8 * X, HID), jnp.float32) + b_ref[0]
    for k in range(9):
        ky, kx = k // 3, k % 3
        patch = padded[ky:ky + 8, kx:kx + X, :].reshape(8 * X, HID)
        acc = acc + jnp.dot(patch, w_ref[k],
                            preferred_element_type=jnp.float32)
    out_ref[...] = acc.reshape(8, X, HID)
    st_ref[0, 0, :] = jnp.sum(acc, axis=0)
    st_ref[0, 1, :] = jnp.sum(acc * acc, axis=0)
    del nstrips


def _conv2(x, sc, sh, w, b):
    gr = Y // 8
    blk = pl.BlockSpec((8, X, HID), lambda i: (i, 0, 0))
    return pl.pallas_call(
        _c2_body,
        grid=(gr,),
        in_specs=[
            pl.BlockSpec((8, X, HID), lambda i: (jnp.maximum(i - 1, 0), 0, 0)),
            blk,
            pl.BlockSpec((8, X, HID),
                         lambda i: (jnp.minimum(i + 1, gr - 1), 0, 0)),
            pl.BlockSpec((1, HID), lambda i: (0, 0)),
            pl.BlockSpec((1, HID), lambda i: (0, 0)),
            pl.BlockSpec((9, HID, HID), lambda i: (0, 0, 0)),
            pl.BlockSpec((1, HID), lambda i: (0, 0)),
        ],
        out_specs=[
            pl.BlockSpec((8, X, HID), lambda i: (i, 0, 0)),
            pl.BlockSpec((1, 2, HID), lambda i: (i, 0, 0)),
        ],
        out_shape=[
            jax.ShapeDtypeStruct((Y, X, HID), jnp.float32),
            jax.ShapeDtypeStruct((gr, 2, HID), jnp.float32),
        ],
        compiler_params=pltpu.CompilerParams(
            dimension_semantics=("arbitrary",)),
    )(x, x, x, sc[None], sh[None], w, b[None])


def _attn_body(x2_ref, sc_ref, sh_ref, bev_ref, winw_ref, winb_ref,
               wq_ref, bq_ref, wk_ref, bk_ref, wv_ref, bv_ref,
               wo_ref, bo_ref, wow_ref, wob_ref, out_ref):
    t = jnp.maximum(x2_ref[...] * sc_ref[0] + sh_ref[0], 0.0)
    t = t.reshape(20, 2, 40, HID).max(axis=1)
    t = t.reshape(20, 20, 2, HID).max(axis=2)
    kv = t.reshape(WS * WS, HID)

    bev = bev_ref[0]
    qw = jnp.dot(bev, winw_ref[...],
                 preferred_element_type=jnp.float32) + winb_ref[0]
    qh = jnp.dot(qw, wq_ref[...], preferred_element_type=jnp.float32) + bq_ref[0]
    kh = jnp.dot(kv, wk_ref[...], preferred_element_type=jnp.float32) + bk_ref[0]
    vh = jnp.dot(kv, wv_ref[...], preferred_element_type=jnp.float32) + bv_ref[0]
    scale = jnp.float32(1.0 / np.sqrt(HD))
    outs = []
    for h in range(NH):
        s = slice(h * HD, (h + 1) * HD)
        a = jax.lax.dot_general(qh[:, s], kh[:, s],
                                (((1,), (1,)), ((), ())),
                                preferred_element_type=jnp.float32) * scale
        a = jax.nn.softmax(a, axis=-1)
        outs.append(jnp.dot(a, vh[:, s], preferred_element_type=jnp.float32))
    o = jnp.concatenate(outs, axis=-1)
    o = jnp.dot(o, wo_ref[...], preferred_element_type=jnp.float32) \
        + bo_ref[0] + qw
    res = jnp.dot(o, wow_ref[...],
                  preferred_element_type=jnp.float32) + wob_ref[0]
    out_ref[0] = res


def _attention(x2, sc, sh, bev_win, win_w, win_b, wq, bq, wk, bk, wv, bv,
               wo, bo, wout_w, wout_b):
    full2 = lambda a, b: pl.BlockSpec((a, b), lambda wy, wx: (0, 0))
    return pl.pallas_call(
        _attn_body,
        grid=(NW, NW),
        in_specs=[
            pl.BlockSpec((2 * WS, 2 * WS, HID), lambda wy, wx: (wy, wx, 0)),
            full2(1, HID), full2(1, HID),
            pl.BlockSpec((1, WS * WS, BZ * BC),
                         lambda wy, wx: (wy * NW + wx, 0, 0)),
            full2(BZ * BC, HID), full2(1, HID),
            full2(HID, HID), full2(1, HID),
            full2(HID, HID), full2(1, HID),
            full2(HID, HID), full2(1, HID),
            full2(HID, HID), full2(1, HID),
            full2(HID, BZ * BC), full2(1, BZ * BC),
        ],
        out_specs=pl.BlockSpec((1, WS * WS, BZ * BC),
                               lambda wy, wx: (wy * NW + wx, 0, 0)),
        out_shape=jax.ShapeDtypeStruct((NW * NW, WS * WS, BZ * BC),
                                       jnp.float32),
        compiler_params=pltpu.CompilerParams(
            dimension_semantics=("parallel", "arbitrary")),
    )(x2, sc[None], sh[None], bev_win, win_w, win_b[None], wq, bq[None],
      wk, bk[None], wv, bv[None], wo, bo[None], wout_w, wout_b[None])


def _finalize_bn(stats, g, b):
    s = jnp.sum(stats[:, 0, :], axis=0)
    sq = jnp.sum(stats[:, 1, :], axis=0)
    m = s / NPIX
    v = sq / NPIX - m * m
    sc = g * jax.lax.rsqrt(v + EPS)
    return sc, b - m * sc


def kernel(bev_feats, prior_feats, prior_voxels_coords, w1, b1, w2, b2,
           conv1_w, conv1_b, bn1_g, bn1_b, conv2_w, conv2_b, bn2_g, bn2_b,
           wq, bq, wk, bk, wv, bv, wo, bo, win_w, win_b, wout_w, wout_b):
    f = _mlp(prior_feats.reshape(N, CIN), w1, b1, w2, b2)

    co = prior_voxels_coords.reshape(N, 3)
    v = (co[:, 1] * X + co[:, 0]) * ZP + co[:, 2]
    vs, ps = jax.lax.sort_key_val(v, jnp.arange(N, dtype=jnp.int32),
                                  is_stable=True)
    packed = jnp.left_shift(vs % SVOX, 17) | ps
    strip = vs // SVOX
    starts = jnp.sum(strip[None, :] < jnp.arange(NSTRIP + 1)[:, None],
                     axis=1).astype(jnp.int32)

    w1z = conv1_w[:, :, 0, 0].reshape(HID, CV, ZP).transpose(2, 1, 0)
    out1, st1 = _scatter_conv1(packed, starts, f.reshape(N // 2, 2 * CV),
                               w1z, conv1_b)
    sc1, sh1 = _finalize_bn(st1, bn1_g, bn1_b)

    w3 = conv2_w.transpose(2, 3, 1, 0).reshape(9, HID, HID)
    out2, st2 = _conv2(out1.reshape(Y, X, HID), sc1, sh1, w3, conv2_b)
    sc2, sh2 = _finalize_bn(st2, bn2_g, bn2_b)

    bev = bev_feats.reshape(BC, H, W, BZ).transpose(1, 2, 3, 0)
    bev = bev.reshape(NW, WS, NW, WS, BZ * BC).transpose(0, 2, 1, 3, 4)
    bev_win = bev.reshape(NW * NW, WS * WS, BZ * BC)

    outw = _attention(out2, sc2, sh2, bev_win, win_w, win_b,
                      wq, bq, wk, bk, wv, bv, wo, bo, wout_w, wout_b)

    out = outw.reshape(NW, NW, WS, WS, BZ * BC).transpose(0, 2, 1, 3, 4)
    out = out.reshape(H, W, BZ, BC).transpose(3, 0, 1, 2)
    return out[None]
